# Initial kernel scaffold; baseline (speedup 1.0000x reference)
import functools

import jax
import jax.numpy as jnp
from jax import lax
from jax.experimental import pallas as pl
from jax.experimental.pallas import tpu as pltpu
from jax.experimental.pallas import tpu_sc as plsc

R = 4096
C = 4096
N_BINS = 128
RB = 256
GRID = R // RB
NW = 32
EPT = (R * C) // NW
CHUNK = 16384
NCHUNK = EPT // CHUNK
LANES = 16
VPC = CHUNK // LANES


def _stats_body(x_ref, v_ref, colmean_ref, scal_ref):
    i = pl.program_id(0)
    x = x_ref[...]
    m = jnp.max(x, axis=1, keepdims=True)
    e = jnp.exp(x - m)
    s = jnp.sum(e, axis=1, keepdims=True)
    v = e / s
    v_ref[...] = v

    csum = jnp.sum(x, axis=0, keepdims=True)
    cmax = jnp.max(1.0 / s)
    cmin = jnp.min(v)
    lane = lax.broadcasted_iota(jnp.int32, (1, N_BINS), 1)
    cur = jnp.where(lane == 0, cmin, cmax)

    @pl.when(i == 0)
    def _():
        colmean_ref[...] = csum
        scal_ref[...] = cur

    @pl.when(i > 0)
    def _():
        colmean_ref[...] = colmean_ref[...] + csum
        prev = scal_ref[...]
        scal_ref[...] = jnp.where(lane == 0, jnp.minimum(prev, cur),
                                  jnp.maximum(prev, cur))

    @pl.when(i == GRID - 1)
    def _():
        colmean_ref[...] = colmean_ref[...] * (1.0 / R)
        vals = scal_ref[...]
        gmin = jnp.min(jnp.where(lane == 0, vals, jnp.inf))
        gmax = jnp.max(jnp.where(lane == 0, -jnp.inf, vals))
        d = (gmax - gmin) / N_BINS
        invd = 1.0 / d
        scal_ref[...] = jnp.where(
            lane == 0, gmin,
            jnp.where(lane == 1, invd, jnp.where(lane == 2, d, gmax)))


_sc_mesh = plsc.VectorSubcoreMesh(core_axis_name="c", subcore_axis_name="s")


@functools.partial(
    pl.kernel,
    mesh=_sc_mesh,
    out_type=jax.ShapeDtypeStruct((NW * N_BINS,), jnp.float32),
    scratch_types=[
        pltpu.VMEM((CHUNK,), jnp.float32),
        pltpu.VMEM((LANES * N_BINS,), jnp.float32),
        pltpu.VMEM((N_BINS,), jnp.float32),
        pltpu.VMEM((LANES,), jnp.float32),
    ],
)
def _hist_kernel(v_hbm, scal_hbm, out_hbm, buf, hist, redu, scal_v):
    wid = lax.axis_index("s") * 2 + lax.axis_index("c")
    base = wid * EPT

    pltpu.sync_copy(scal_hbm.at[pl.ds(0, LANES)], scal_v)
    gmin = scal_v[0]
    invd = scal_v[1]
    c0 = gmin * invd

    zeros = jnp.zeros((LANES,), jnp.float32)
    for k in range(N_BINS):
        hist[pl.ds(k * LANES, LANES)] = zeros

    lane_base = lax.iota(jnp.int32, LANES) * N_BINS

    def chunk_body(k, carry):
        pltpu.sync_copy(v_hbm.at[pl.ds(base + k * CHUNK, CHUNK)], buf)

        def vec_body(j, c):
            vec = buf[pl.ds(j * LANES, LANES)]
            t = vec * invd - c0
            idx = t.astype(jnp.int32)
            vals = jnp.where(idx < N_BINS, vec, 0.0)
            addr = lane_base + jnp.minimum(idx, N_BINS - 1)
            plsc.addupdate_scatter(hist, [addr], vals)
            return c

        return lax.fori_loop(0, VPC, vec_body, carry, unroll=8)

    lax.fori_loop(0, NCHUNK, chunk_body, 0)

    for g in range(N_BINS // LANES):
        acc = jnp.zeros((LANES,), jnp.float32)
        for l in range(LANES):
            acc = acc + hist[pl.ds(l * N_BINS + g * LANES, LANES)]
        redu[pl.ds(g * LANES, LANES)] = acc

    pltpu.sync_copy(redu, out_hbm.at[pl.ds(wid * N_BINS, N_BINS)])


def _assemble_body(colmean_ref, parts_ref, out_ref):
    out_ref[:, :C] = colmean_ref[...]
    out_ref[:, C:] = jnp.sum(parts_ref[...], axis=0, keepdims=True)


def kernel(seq):
    x = seq[0]
    v, colmean, scal = pl.pallas_call(
        _stats_body,
        grid=(GRID,),
        in_specs=[pl.BlockSpec((RB, C), lambda i: (i, 0))],
        out_specs=[
            pl.BlockSpec((RB, C), lambda i: (i, 0)),
            pl.BlockSpec((1, C), lambda i: (0, 0)),
            pl.BlockSpec((1, N_BINS), lambda i: (0, 0)),
        ],
        out_shape=[
            jax.ShapeDtypeStruct((R, C), jnp.float32),
            jax.ShapeDtypeStruct((1, C), jnp.float32),
            jax.ShapeDtypeStruct((1, N_BINS), jnp.float32),
        ],
    )(x)

    parts = _hist_kernel(v.reshape(-1), scal.reshape(-1))

    out = pl.pallas_call(
        _assemble_body,
        out_shape=jax.ShapeDtypeStruct((1, C + N_BINS), jnp.float32),
    )(colmean, parts.reshape(NW, N_BINS))
    return out


# speedup vs baseline: 26.5592x; 26.5592x over previous
import functools

import jax
import jax.numpy as jnp
from jax import lax
from jax.experimental import pallas as pl
from jax.experimental.pallas import tpu as pltpu
from jax.experimental.pallas import tpu_sc as plsc

R = 4096
C = 4096
N_BINS = 128
RB = 256
GRID = R // RB
NW = 32
EPT = (R * C) // NW
CHUNK = 16384
NCHUNK = EPT // CHUNK
LANES = 16
VPC = CHUNK // LANES


def _stats_body(x_ref, v_ref, colmean_ref, scal_ref):
    i = pl.program_id(0)
    x = x_ref[...]
    m = jnp.max(x, axis=1, keepdims=True)
    e = jnp.exp(x - m)
    s = jnp.sum(e, axis=1, keepdims=True)
    v = e / s
    v_ref[...] = v

    csum = jnp.sum(x, axis=0, keepdims=True)
    cmax = jnp.max(1.0 / s)
    cmin = jnp.min(v)
    lane = lax.broadcasted_iota(jnp.int32, (1, N_BINS), 1)
    cur = jnp.where(lane == 0, cmin, cmax)

    @pl.when(i == 0)
    def _():
        colmean_ref[...] = csum
        scal_ref[...] = cur

    @pl.when(i > 0)
    def _():
        colmean_ref[...] = colmean_ref[...] + csum
        prev = scal_ref[...]
        scal_ref[...] = jnp.where(lane == 0, jnp.minimum(prev, cur),
                                  jnp.maximum(prev, cur))

    @pl.when(i == GRID - 1)
    def _():
        colmean_ref[...] = colmean_ref[...] * (1.0 / R)
        vals = scal_ref[...]
        gmin = jnp.min(jnp.where(lane == 0, vals, jnp.inf))
        gmax = jnp.max(jnp.where(lane == 0, -jnp.inf, vals))
        d = (gmax - gmin) / N_BINS
        invd = 1.0 / d
        scal_ref[...] = jnp.where(
            lane == 0, gmin,
            jnp.where(lane == 1, invd, jnp.where(lane == 2, d, gmax)))


def _hist_body(v_hbm, scal_hbm, out_hbm, buf, hist, redu, scal_v):
    wid = lax.axis_index("s") * 2 + lax.axis_index("c")
    base = wid * EPT

    pltpu.sync_copy(scal_hbm.at[pl.ds(0, LANES)], scal_v)
    sv = scal_v[pl.ds(0, LANES)]
    gmin = sv[0]
    invd = sv[1]
    c0 = gmin * invd

    zeros = jnp.zeros((LANES,), jnp.float32)
    for k in range(N_BINS):
        hist[pl.ds(k * LANES, LANES)] = zeros

    lane_base = lax.iota(jnp.int32, LANES) * N_BINS

    def chunk_body(k, carry):
        pltpu.sync_copy(v_hbm.at[pl.ds(base + k * CHUNK, CHUNK)], buf)

        def vec_body(j, c):
            vec = buf[pl.ds(j * LANES, LANES)]
            t = vec * invd - c0
            idx = t.astype(jnp.int32)
            vals = jnp.where(idx < N_BINS, vec, 0.0)
            addr = lane_base + jnp.minimum(idx, N_BINS - 1)
            plsc.addupdate_scatter(hist, [addr], vals)
            return c

        return lax.fori_loop(0, VPC, vec_body, carry, unroll=8)

    lax.fori_loop(0, NCHUNK, chunk_body, 0)

    for g in range(N_BINS // LANES):
        acc = jnp.zeros((LANES,), jnp.float32)
        for l in range(LANES):
            acc = acc + hist[pl.ds(l * N_BINS + g * LANES, LANES)]
        redu[pl.ds(g * LANES, LANES)] = acc

    pltpu.sync_copy(redu, out_hbm.at[pl.ds(wid * N_BINS, N_BINS)])


def _assemble_body(colmean_ref, parts_ref, out_ref):
    out_ref[:, :C] = colmean_ref[...]
    out_ref[:, C:] = jnp.sum(parts_ref[...], axis=0, keepdims=True)


def kernel(seq):
    x = seq[0]
    v, colmean, scal = pl.pallas_call(
        _stats_body,
        grid=(GRID,),
        in_specs=[pl.BlockSpec((RB, C), lambda i: (i, 0))],
        out_specs=[
            pl.BlockSpec((RB, C), lambda i: (i, 0)),
            pl.BlockSpec((1, C), lambda i: (0, 0)),
            pl.BlockSpec((1, N_BINS), lambda i: (0, 0)),
        ],
        out_shape=[
            jax.ShapeDtypeStruct((R, C), jnp.float32),
            jax.ShapeDtypeStruct((1, C), jnp.float32),
            jax.ShapeDtypeStruct((1, N_BINS), jnp.float32),
        ],
    )(x)

    hist_kernel = pl.kernel(
        _hist_body,
        mesh=plsc.VectorSubcoreMesh(core_axis_name="c", subcore_axis_name="s"),
        out_type=jax.ShapeDtypeStruct((NW * N_BINS,), jnp.float32),
        scratch_types=[
            pltpu.VMEM((CHUNK,), jnp.float32),
            pltpu.VMEM((LANES * N_BINS,), jnp.float32),
            pltpu.VMEM((N_BINS,), jnp.float32),
            pltpu.VMEM((LANES,), jnp.float32),
        ],
        compiler_params=pltpu.CompilerParams(needs_layout_passes=False),
    )
    parts = hist_kernel(v.reshape(-1), scal.reshape(-1))

    out = pl.pallas_call(
        _assemble_body,
        out_shape=jax.ShapeDtypeStruct((1, C + N_BINS), jnp.float32),
    )(colmean, parts.reshape(NW, N_BINS))
    return out


# speedup vs baseline: 89.2857x; 3.3618x over previous
import functools

import jax
import jax.numpy as jnp
from jax import lax
from jax.experimental import pallas as pl
from jax.experimental.pallas import tpu as pltpu
from jax.experimental.pallas import tpu_sc as plsc

R = 4096
C = 4096
N_BINS = 128
RB = 256
GRID = R // RB
NW = 32
EPT = (R * C) // NW
CHUNK = 16384
NCHUNK = EPT // CHUNK
LANES = 16
VPC = CHUNK // LANES


def _stats_body(x_ref, v_ref, colmean_ref, scal_ref):
    i = pl.program_id(0)
    x = x_ref[...]
    m = jnp.max(x, axis=1, keepdims=True)
    e = jnp.exp(x - m)
    s = jnp.sum(e, axis=1, keepdims=True)
    v = e / s
    v_ref[...] = v

    csum = jnp.sum(x, axis=0, keepdims=True)
    cmax = jnp.max(1.0 / s)
    cmin = jnp.min(v)
    lane = lax.broadcasted_iota(jnp.int32, (1, N_BINS), 1)
    cur = jnp.where(lane == 0, cmin, cmax)

    @pl.when(i == 0)
    def _():
        colmean_ref[...] = csum
        scal_ref[...] = cur

    @pl.when(i > 0)
    def _():
        colmean_ref[...] = colmean_ref[...] + csum
        prev = scal_ref[...]
        scal_ref[...] = jnp.where(lane == 0, jnp.minimum(prev, cur),
                                  jnp.maximum(prev, cur))

    @pl.when(i == GRID - 1)
    def _():
        colmean_ref[...] = colmean_ref[...] * (1.0 / R)
        vals = scal_ref[...]
        gmin = jnp.min(jnp.where(lane == 0, vals, jnp.inf))
        gmax = jnp.max(jnp.where(lane == 0, -jnp.inf, vals))
        d = (gmax - gmin) / N_BINS
        invd = 1.0 / d
        scal_ref[...] = jnp.where(
            lane == 0, gmin,
            jnp.where(lane == 1, invd, jnp.where(lane == 2, d, gmax)))


NV = 8
HSTRIDE = N_BINS + 1
HCOPY = HSTRIDE * LANES


def _hist_body(v_hbm, scal_hbm, out_hbm, buf0, buf1, hist, redu, scal_v,
               sem0, sem1):
    wid = lax.axis_index("s") * 2 + lax.axis_index("c")
    base = wid * EPT

    pltpu.sync_copy(scal_hbm.at[pl.ds(0, LANES)], scal_v)
    sv = scal_v[pl.ds(0, LANES)]
    gmin = sv[0]
    invd = sv[1]
    c0 = gmin * invd

    zeros = jnp.zeros((LANES,), jnp.float32)

    def zero_body(z, c):
        hist[pl.ds(z * LANES, LANES)] = zeros
        return c

    lax.fori_loop(0, NV * HCOPY // LANES, zero_body, 0)

    lane_bases = [lax.iota(jnp.int32, LANES) * HSTRIDE + h * HCOPY
                  for h in range(NV)]

    def process(b):
        def vblock(j, c):
            vecs = [b[pl.ds((j * NV + i) * LANES, LANES)] for i in range(NV)]
            for i in range(NV):
                idx = (vecs[i] * invd - c0).astype(jnp.int32)
                plsc.addupdate_scatter(hist, [lane_bases[i] + idx], vecs[i])
            return c

        lax.fori_loop(0, VPC // NV, vblock, 0)

    def start(k, b, sem):
        pltpu.make_async_copy(v_hbm.at[pl.ds(base + k * CHUNK, CHUNK)],
                              b, sem).start()

    def wait(b, sem):
        pltpu.make_async_copy(v_hbm.at[pl.ds(base, CHUNK)], b, sem).wait()

    start(0, buf0, sem0)
    start(1, buf1, sem1)

    def chunk_body(m, c):
        wait(buf0, sem0)
        process(buf0)

        @pl.when(m < NCHUNK // 2 - 1)
        def _():
            start(2 * m + 2, buf0, sem0)

        wait(buf1, sem1)
        process(buf1)

        @pl.when(m < NCHUNK // 2 - 1)
        def _():
            start(2 * m + 3, buf1, sem1)

        return c

    lax.fori_loop(0, NCHUNK // 2, chunk_body, 0)

    for g in range(N_BINS // LANES):
        acc = jnp.zeros((LANES,), jnp.float32)
        for h in range(NV):
            for l in range(LANES):
                acc = acc + hist[pl.ds(h * HCOPY + l * HSTRIDE + g * LANES,
                                       LANES)]
        redu[pl.ds(g * LANES, LANES)] = acc

    pltpu.sync_copy(redu, out_hbm.at[pl.ds(wid * N_BINS, N_BINS)])


def _assemble_body(colmean_ref, parts_ref, out_ref):
    out_ref[:, :C] = colmean_ref[...]
    out_ref[:, C:] = jnp.sum(parts_ref[...], axis=0, keepdims=True)


def kernel(seq):
    x = seq[0]
    v, colmean, scal = pl.pallas_call(
        _stats_body,
        grid=(GRID,),
        in_specs=[pl.BlockSpec((RB, C), lambda i: (i, 0))],
        out_specs=[
            pl.BlockSpec((RB, C), lambda i: (i, 0)),
            pl.BlockSpec((1, C), lambda i: (0, 0)),
            pl.BlockSpec((1, N_BINS), lambda i: (0, 0)),
        ],
        out_shape=[
            jax.ShapeDtypeStruct((R, C), jnp.float32),
            jax.ShapeDtypeStruct((1, C), jnp.float32),
            jax.ShapeDtypeStruct((1, N_BINS), jnp.float32),
        ],
    )(x)

    hist_kernel = pl.kernel(
        _hist_body,
        mesh=plsc.VectorSubcoreMesh(core_axis_name="c", subcore_axis_name="s"),
        out_type=jax.ShapeDtypeStruct((NW * N_BINS,), jnp.float32),
        scratch_types=[
            pltpu.VMEM((CHUNK,), jnp.float32),
            pltpu.VMEM((CHUNK,), jnp.float32),
            pltpu.VMEM((NV * HCOPY,), jnp.float32),
            pltpu.VMEM((N_BINS,), jnp.float32),
            pltpu.VMEM((LANES,), jnp.float32),
            pltpu.SemaphoreType.DMA,
            pltpu.SemaphoreType.DMA,
        ],
        compiler_params=pltpu.CompilerParams(needs_layout_passes=False),
    )
    parts = hist_kernel(v.reshape(-1), scal.reshape(-1))

    out = pl.pallas_call(
        _assemble_body,
        out_shape=jax.ShapeDtypeStruct((1, C + N_BINS), jnp.float32),
    )(colmean, parts.reshape(NW, N_BINS))
    return out


# speedup vs baseline: 105.1012x; 1.1771x over previous
import functools

import jax
import jax.numpy as jnp
from jax import lax
from jax.experimental import pallas as pl
from jax.experimental.pallas import tpu as pltpu
from jax.experimental.pallas import tpu_sc as plsc

R = 4096
C = 4096
N_BINS = 128
RB = 256
GRID = R // RB
NW = 32
EPT = (R * C) // NW
CHUNK = 16384
NCHUNK = EPT // CHUNK
LANES = 16
VPC = CHUNK // LANES


def _stats_body(x_ref, v_ref, colmean_ref, scal_ref):
    i = pl.program_id(0)
    x = x_ref[...]
    m = jnp.max(x, axis=1, keepdims=True)
    e = jnp.exp(x - m)
    s = jnp.sum(e, axis=1, keepdims=True)
    r = 1.0 / s
    v = e * r
    v_ref[...] = v

    csum = jnp.sum(x, axis=0, keepdims=True)
    cmax = jnp.max(r)
    cmin = jnp.min(v)
    lane = lax.broadcasted_iota(jnp.int32, (1, N_BINS), 1)
    cur = jnp.where(lane == 0, cmin, cmax)

    @pl.when(i == 0)
    def _():
        colmean_ref[...] = csum
        scal_ref[...] = cur

    @pl.when(i > 0)
    def _():
        colmean_ref[...] = colmean_ref[...] + csum
        prev = scal_ref[...]
        scal_ref[...] = jnp.where(lane == 0, jnp.minimum(prev, cur),
                                  jnp.maximum(prev, cur))

    @pl.when(i == GRID - 1)
    def _():
        colmean_ref[...] = colmean_ref[...] * (1.0 / R)
        vals = scal_ref[...]
        gmin = jnp.min(jnp.where(lane == 0, vals, jnp.inf))
        gmax = jnp.max(jnp.where(lane == 0, -jnp.inf, vals))
        d = (gmax - gmin) / N_BINS
        invd = 1.0 / d
        scal_ref[...] = jnp.where(
            lane == 0, gmin,
            jnp.where(lane == 1, invd, jnp.where(lane == 2, d, gmax)))


NV = 16
HSTRIDE = N_BINS + 3
HCOPY = HSTRIDE * LANES
MAGIC = float(3 * 2 ** 22)


def _hist_body(v_hbm, scal_hbm, out_hbm, buf0, buf1, hist, redu, scal_v,
               sem0, sem1):
    wid = lax.axis_index("s") * 2 + lax.axis_index("c")
    base = wid * EPT

    pltpu.sync_copy(scal_hbm.at[pl.ds(0, LANES)], scal_v)
    sv = scal_v[pl.ds(0, LANES)]
    gmin = sv[0]
    invd = sv[1]
    c1 = gmin * invd + 0.5

    zeros = jnp.zeros((LANES,), jnp.float32)

    def zero_body(z, c):
        hist[pl.ds(z * LANES, LANES)] = zeros
        return c

    lax.fori_loop(0, NV * HCOPY // LANES, zero_body, 0)

    bias = 0x4B400000
    lane_bias = [lax.iota(jnp.int32, LANES) * HSTRIDE + (h * HCOPY + 1 - bias)
                 for h in range(NV)]

    def process(b):
        def vblock(j, c):
            vecs = [b[pl.ds((j * NV + i) * LANES, LANES)] for i in range(NV)]
            for i in range(NV):
                u = (vecs[i] * invd - c1) + MAGIC
                addr = plsc.bitcast(u, jnp.int32) + lane_bias[i]
                plsc.addupdate_scatter(hist, [addr], vecs[i])
            return c

        lax.fori_loop(0, VPC // NV, vblock, 0)

    def start(k, b, sem):
        pltpu.make_async_copy(v_hbm.at[pl.ds(base + k * CHUNK, CHUNK)],
                              b, sem).start()

    def wait(b, sem):
        pltpu.make_async_copy(v_hbm.at[pl.ds(base, CHUNK)], b, sem).wait()

    start(0, buf0, sem0)
    start(1, buf1, sem1)

    def chunk_body(m, c):
        wait(buf0, sem0)
        process(buf0)

        @pl.when(m < NCHUNK // 2 - 1)
        def _():
            start(2 * m + 2, buf0, sem0)

        wait(buf1, sem1)
        process(buf1)

        @pl.when(m < NCHUNK // 2 - 1)
        def _():
            start(2 * m + 3, buf1, sem1)

        return c

    lax.fori_loop(0, NCHUNK // 2, chunk_body, 0)

    def red_body(i, accs):
        off = i * HSTRIDE + 1
        return tuple(a + hist[pl.ds(off + g * LANES, LANES)]
                     for g, a in enumerate(accs))

    accs = lax.fori_loop(0, NV * LANES, red_body,
                         tuple(jnp.zeros((LANES,), jnp.float32)
                               for _ in range(N_BINS // LANES)))
    for g in range(N_BINS // LANES):
        redu[pl.ds(g * LANES, LANES)] = accs[g]

    pltpu.sync_copy(redu, out_hbm.at[pl.ds(wid * N_BINS, N_BINS)])


def _assemble_body(colmean_ref, parts_ref, out_ref):
    out_ref[:, :C] = colmean_ref[...]
    out_ref[:, C:] = jnp.sum(parts_ref[...], axis=0, keepdims=True)


def kernel(seq):
    x = seq[0]
    v, colmean, scal = pl.pallas_call(
        _stats_body,
        grid=(GRID,),
        in_specs=[pl.BlockSpec((RB, C), lambda i: (i, 0))],
        out_specs=[
            pl.BlockSpec((RB, C), lambda i: (i, 0)),
            pl.BlockSpec((1, C), lambda i: (0, 0)),
            pl.BlockSpec((1, N_BINS), lambda i: (0, 0)),
        ],
        out_shape=[
            jax.ShapeDtypeStruct((R, C), jnp.float32),
            jax.ShapeDtypeStruct((1, C), jnp.float32),
            jax.ShapeDtypeStruct((1, N_BINS), jnp.float32),
        ],
    )(x)

    hist_kernel = pl.kernel(
        _hist_body,
        mesh=plsc.VectorSubcoreMesh(core_axis_name="c", subcore_axis_name="s"),
        out_type=jax.ShapeDtypeStruct((NW * N_BINS,), jnp.float32),
        scratch_types=[
            pltpu.VMEM((CHUNK,), jnp.float32),
            pltpu.VMEM((CHUNK,), jnp.float32),
            pltpu.VMEM((NV * HCOPY,), jnp.float32),
            pltpu.VMEM((N_BINS,), jnp.float32),
            pltpu.VMEM((LANES,), jnp.float32),
            pltpu.SemaphoreType.DMA,
            pltpu.SemaphoreType.DMA,
        ],
        compiler_params=pltpu.CompilerParams(needs_layout_passes=False),
    )
    parts = hist_kernel(v.reshape(-1), scal.reshape(-1))

    out = pl.pallas_call(
        _assemble_body,
        out_shape=jax.ShapeDtypeStruct((1, C + N_BINS), jnp.float32),
    )(colmean, parts.reshape(NW, N_BINS))
    return out


# speedup vs baseline: 143.6191x; 1.3665x over previous
import functools

import jax
import jax.numpy as jnp
from jax import lax
from jax.experimental import pallas as pl
from jax.experimental.pallas import tpu as pltpu
from jax.experimental.pallas import tpu_sc as plsc

R = 4096
C = 4096
N_BINS = 128
RB = 256
GRID = R // RB
NW = 32
EPT = (R * C) // NW
CHUNK = 16384
NCHUNK = EPT // CHUNK
LANES = 16
VPC = CHUNK // LANES


def _stats_body(x_ref, v_ref, colmean_ref, scal_ref):
    i = pl.program_id(0)
    x = x_ref[...]
    m = jnp.max(x, axis=1, keepdims=True)
    e = jnp.exp(x - m)
    s = jnp.sum(e, axis=1, keepdims=True)
    r = 1.0 / s
    v = e * r
    v_ref[...] = v

    csum = jnp.sum(x, axis=0, keepdims=True)
    cmax = jnp.max(r)
    cmin = jnp.min(v)
    lane = lax.broadcasted_iota(jnp.int32, (1, N_BINS), 1)
    cur = jnp.where(lane == 0, cmin, cmax)

    @pl.when(i == 0)
    def _():
        colmean_ref[...] = csum
        scal_ref[...] = cur

    @pl.when(i > 0)
    def _():
        colmean_ref[...] = colmean_ref[...] + csum
        prev = scal_ref[...]
        scal_ref[...] = jnp.where(lane == 0, jnp.minimum(prev, cur),
                                  jnp.maximum(prev, cur))

    @pl.when(i == GRID - 1)
    def _():
        colmean_ref[...] = colmean_ref[...] * (1.0 / R)
        vals = scal_ref[...]
        gmin = jnp.min(jnp.where(lane == 0, vals, jnp.inf))
        gmax = jnp.max(jnp.where(lane == 0, -jnp.inf, vals))
        d = (gmax - gmin) / N_BINS
        invd = 1.0 / d
        scal_ref[...] = jnp.where(
            lane == 0, gmin,
            jnp.where(lane == 1, invd, jnp.where(lane == 2, d, gmax)))


NV = 16
HSTRIDE = N_BINS + 3
HCOPY = HSTRIDE * LANES
MAGIC = float(3 * 2 ** 22)


CROWS = 8
RPT = R // NW
NRCHUNK = RPT // CROWS
VPRC = CROWS * C // LANES


def _hist_body(v_hbm, scal_hbm, out_hbm, buf0, buf1, hist, redu, scal_v,
               sem0, sem1):
    wid = lax.axis_index("s") * 2 + lax.axis_index("c")
    row0 = wid * RPT

    pltpu.sync_copy(scal_hbm.at[pl.ds(0, LANES)], scal_v)
    sv = scal_v[pl.ds(0, LANES)]
    gmin = sv[0]
    invd = sv[1]
    c1 = gmin * invd + 0.5

    zeros = jnp.zeros((LANES,), jnp.float32)

    def zero_body(z, c):
        hist[pl.ds(z * LANES, LANES)] = zeros
        return c

    lax.fori_loop(0, NV * HCOPY // LANES, zero_body, 0)

    bias = 0x4B400000
    lane_bias = [lax.iota(jnp.int32, LANES) * HSTRIDE + (h * HCOPY + 1 - bias)
                 for h in range(NV)]

    def process(b):
        def vblock(j, c):
            r = lax.shift_right_logical(j, 4)
            cb = lax.bitwise_and(j, 15) * (LANES * LANES)
            vecs = [b[r, pl.ds(cb + i * LANES, LANES)] for i in range(NV)]
            for i in range(NV):
                u = (vecs[i] * invd - c1) + MAGIC
                addr = plsc.bitcast(u, jnp.int32) + lane_bias[i]
                plsc.addupdate_scatter(hist, [addr], vecs[i])
            return c

        lax.fori_loop(0, VPRC // NV, vblock, 0)

    def start(k, b, sem):
        pltpu.make_async_copy(v_hbm.at[pl.ds(row0 + k * CROWS, CROWS)],
                              b, sem).start()

    def wait(b, sem):
        pltpu.make_async_copy(v_hbm.at[pl.ds(row0, CROWS)], b, sem).wait()

    start(0, buf0, sem0)
    start(1, buf1, sem1)

    def chunk_body(m, c):
        wait(buf0, sem0)
        process(buf0)

        @pl.when(m < NRCHUNK // 2 - 1)
        def _():
            start(2 * m + 2, buf0, sem0)

        wait(buf1, sem1)
        process(buf1)

        @pl.when(m < NRCHUNK // 2 - 1)
        def _():
            start(2 * m + 3, buf1, sem1)

        return c

    lax.fori_loop(0, NRCHUNK // 2, chunk_body, 0)

    def red_body(i, accs):
        off = i * HSTRIDE + 1
        return tuple(a + hist[pl.ds(off + g * LANES, LANES)]
                     for g, a in enumerate(accs))

    accs = lax.fori_loop(0, NV * LANES, red_body,
                         tuple(jnp.zeros((LANES,), jnp.float32)
                               for _ in range(N_BINS // LANES)))
    for g in range(N_BINS // LANES):
        redu[pl.ds(g * LANES, LANES)] = accs[g]

    pltpu.sync_copy(redu, out_hbm.at[pl.ds(wid * N_BINS, N_BINS)])


def _assemble_body(colmean_ref, parts_ref, out_ref):
    out_ref[:, :C] = colmean_ref[...]
    out_ref[:, C:] = jnp.sum(parts_ref[...], axis=0, keepdims=True)


def kernel(seq):
    x = seq[0]
    v, colmean, scal = pl.pallas_call(
        _stats_body,
        grid=(GRID,),
        in_specs=[pl.BlockSpec((RB, C), lambda i: (i, 0))],
        out_specs=[
            pl.BlockSpec((RB, C), lambda i: (i, 0)),
            pl.BlockSpec((1, C), lambda i: (0, 0)),
            pl.BlockSpec((1, N_BINS), lambda i: (0, 0)),
        ],
        out_shape=[
            jax.ShapeDtypeStruct((R, C), jnp.float32),
            jax.ShapeDtypeStruct((1, C), jnp.float32),
            jax.ShapeDtypeStruct((1, N_BINS), jnp.float32),
        ],
    )(x)

    hist_kernel = pl.kernel(
        _hist_body,
        mesh=plsc.VectorSubcoreMesh(core_axis_name="c", subcore_axis_name="s"),
        out_type=jax.ShapeDtypeStruct((NW * N_BINS,), jnp.float32),
        scratch_types=[
            pltpu.VMEM((CROWS, C), jnp.float32),
            pltpu.VMEM((CROWS, C), jnp.float32),
            pltpu.VMEM((NV * HCOPY,), jnp.float32),
            pltpu.VMEM((N_BINS,), jnp.float32),
            pltpu.VMEM((LANES,), jnp.float32),
            pltpu.SemaphoreType.DMA,
            pltpu.SemaphoreType.DMA,
        ],
        compiler_params=pltpu.CompilerParams(needs_layout_passes=False,
                                             use_tc_tiling_on_sc=True),
    )
    parts = hist_kernel(v, scal.reshape(-1))

    out = pl.pallas_call(
        _assemble_body,
        out_shape=jax.ShapeDtypeStruct((1, C + N_BINS), jnp.float32),
    )(colmean, parts.reshape(NW, N_BINS))
    return out


# speedup vs baseline: 146.2884x; 1.0186x over previous
import functools

import jax
import jax.numpy as jnp
from jax import lax
from jax.experimental import pallas as pl
from jax.experimental.pallas import tpu as pltpu
from jax.experimental.pallas import tpu_sc as plsc

R = 4096
C = 4096
N_BINS = 128
RB = 256
GRID = R // RB
NW = 32
EPT = (R * C) // NW
CHUNK = 16384
NCHUNK = EPT // CHUNK
LANES = 16
VPC = CHUNK // LANES


def _stats_body(x_ref, v_ref, colmean_ref, scal_ref):
    i = pl.program_id(0)
    x = x_ref[...]
    m = jnp.max(x, axis=1, keepdims=True)
    e = jnp.exp(x - m)
    s = jnp.sum(e, axis=1, keepdims=True)
    r = 1.0 / s
    v = e * r
    v_ref[...] = v

    csum = jnp.sum(x, axis=0, keepdims=True)
    cmax = jnp.max(r)
    cmin = jnp.min(v)
    lane = lax.broadcasted_iota(jnp.int32, (1, N_BINS), 1)
    cur = jnp.where(lane == 0, cmin, cmax)

    @pl.when(i == 0)
    def _():
        colmean_ref[...] = csum
        scal_ref[...] = cur

    @pl.when(i > 0)
    def _():
        colmean_ref[...] = colmean_ref[...] + csum
        prev = scal_ref[...]
        scal_ref[...] = jnp.where(lane == 0, jnp.minimum(prev, cur),
                                  jnp.maximum(prev, cur))

    @pl.when(i == GRID - 1)
    def _():
        colmean_ref[...] = colmean_ref[...] * (1.0 / R)
        vals = scal_ref[...]
        gmin = jnp.min(jnp.where(lane == 0, vals, jnp.inf))
        gmax = jnp.max(jnp.where(lane == 0, -jnp.inf, vals))
        d = (gmax - gmin) / N_BINS
        invd = 1.0 / d
        scal_ref[...] = jnp.where(
            lane == 0, gmin,
            jnp.where(lane == 1, invd, jnp.where(lane == 2, d, gmax)))


NV = 16
HSTRIDE = N_BINS + 3
HCOPY = HSTRIDE * LANES
MAGIC = float(3 * 2 ** 22)


CROWS = 8
RPT = R // NW
NRCHUNK = RPT // CROWS
VPRC = CROWS * C // LANES


def _hist_body(v_hbm, scal_hbm, out_hbm, buf0, buf1, hist, redu, scal_v,
               sem0, sem1):
    wid = lax.axis_index("s") * 2 + lax.axis_index("c")
    row0 = wid * RPT

    pltpu.sync_copy(scal_hbm.at[pl.ds(0, LANES)], scal_v)
    sv = scal_v[pl.ds(0, LANES)]
    gmin = sv[0]
    invd = sv[1]
    c1 = gmin * invd + 0.5

    zeros = jnp.zeros((LANES,), jnp.float32)

    def zero_body(z, c):
        hist[pl.ds(z * LANES, LANES)] = zeros
        return c

    lax.fori_loop(0, NV * HCOPY // LANES, zero_body, 0)

    bias = jnp.int32(-0x4B400000)
    lane_f = lax.iota(jnp.int32, LANES).astype(jnp.float32) * float(HSTRIDE)
    magic_vec = [lane_f + (MAGIC + h * HCOPY + 1) for h in range(NV)]

    def process(b):
        def vblock(j):
            r = lax.shift_right_logical(j, 4)
            cb = lax.bitwise_and(j, 15) * (LANES * LANES)
            vecs = [b[r, pl.ds(cb + i * LANES, LANES)] for i in range(NV)]
            for i in range(NV):
                u = (vecs[i] * invd - c1) + magic_vec[i]
                addr = plsc.bitcast(u, jnp.int32) + bias
                plsc.addupdate_scatter(hist, [addr], vecs[i])

        plsc.parallel_loop(0, VPRC // NV, 1)(vblock)

    def start(k, b, sem):
        pltpu.make_async_copy(v_hbm.at[pl.ds(row0 + k * CROWS, CROWS)],
                              b, sem).start()

    def wait(b, sem):
        pltpu.make_async_copy(v_hbm.at[pl.ds(row0, CROWS)], b, sem).wait()

    start(0, buf0, sem0)
    start(1, buf1, sem1)

    def chunk_body(m, c):
        wait(buf0, sem0)
        process(buf0)

        @pl.when(m < NRCHUNK // 2 - 1)
        def _():
            start(2 * m + 2, buf0, sem0)

        wait(buf1, sem1)
        process(buf1)

        @pl.when(m < NRCHUNK // 2 - 1)
        def _():
            start(2 * m + 3, buf1, sem1)

        return c

    lax.fori_loop(0, NRCHUNK // 2, chunk_body, 0)

    def red_body(i, accs):
        off = i * HSTRIDE + 1
        return tuple(a + hist[pl.ds(off + g * LANES, LANES)]
                     for g, a in enumerate(accs))

    accs = lax.fori_loop(0, NV * LANES, red_body,
                         tuple(jnp.zeros((LANES,), jnp.float32)
                               for _ in range(N_BINS // LANES)))
    for g in range(N_BINS // LANES):
        redu[pl.ds(g * LANES, LANES)] = accs[g]

    pltpu.sync_copy(redu, out_hbm.at[pl.ds(wid * N_BINS, N_BINS)])


def _assemble_body(colmean_ref, parts_ref, out_ref):
    out_ref[:, :C] = colmean_ref[...]
    out_ref[:, C:] = jnp.sum(parts_ref[...], axis=0, keepdims=True)


def kernel(seq):
    x = seq[0]
    v, colmean, scal = pl.pallas_call(
        _stats_body,
        grid=(GRID,),
        in_specs=[pl.BlockSpec((RB, C), lambda i: (i, 0))],
        out_specs=[
            pl.BlockSpec((RB, C), lambda i: (i, 0)),
            pl.BlockSpec((1, C), lambda i: (0, 0)),
            pl.BlockSpec((1, N_BINS), lambda i: (0, 0)),
        ],
        out_shape=[
            jax.ShapeDtypeStruct((R, C), jnp.float32),
            jax.ShapeDtypeStruct((1, C), jnp.float32),
            jax.ShapeDtypeStruct((1, N_BINS), jnp.float32),
        ],
    )(x)

    hist_kernel = pl.kernel(
        _hist_body,
        mesh=plsc.VectorSubcoreMesh(core_axis_name="c", subcore_axis_name="s"),
        out_type=jax.ShapeDtypeStruct((NW * N_BINS,), jnp.float32),
        scratch_types=[
            pltpu.VMEM((CROWS, C), jnp.float32),
            pltpu.VMEM((CROWS, C), jnp.float32),
            pltpu.VMEM((NV * HCOPY,), jnp.float32),
            pltpu.VMEM((N_BINS,), jnp.float32),
            pltpu.VMEM((LANES,), jnp.float32),
            pltpu.SemaphoreType.DMA,
            pltpu.SemaphoreType.DMA,
        ],
        compiler_params=pltpu.CompilerParams(needs_layout_passes=False,
                                             use_tc_tiling_on_sc=True),
    )
    parts = hist_kernel(v, scal.reshape(-1))

    out = pl.pallas_call(
        _assemble_body,
        out_shape=jax.ShapeDtypeStruct((1, C + N_BINS), jnp.float32),
    )(colmean, parts.reshape(NW, N_BINS))
    return out
